# baseline (speedup 1.0000x reference)
import functools

import jax
import jax.numpy as jnp
from jax.experimental import pallas as pl

IMG_SIZE = 512.0


def _decode_kernel(x_ref, anch_ref, o_ref, *, G, C, BB):
    stride = IMG_SIZE / G
    gx = jax.lax.broadcasted_iota(jnp.int32, (G, G), 1).astype(jnp.float32)
    gy = jax.lax.broadcasted_iota(jnp.int32, (G, G), 0).astype(jnp.float32)
    for bb in range(BB):
        for c in range(C):
            a, f = c // 5, c % 5
            v = x_ref[bb, c]
            if f < 3:
                s = jax.nn.sigmoid(v)
                if f == 0:
                    r = s
                elif f == 1:
                    r = (s + gx) * stride
                else:
                    r = (s + gy) * stride
            else:
                r = jnp.exp(v) * anch_ref[a, f - 3]
            o_ref[bb, a, f] = r


@jax.jit
def kernel(y_pred, anchors):
    B, G, _, C = y_pred.shape
    A = anchors.shape[0]
    x_t = jnp.transpose(y_pred, (0, 3, 1, 2))
    BB = 8
    out = pl.pallas_call(
        functools.partial(_decode_kernel, G=G, C=C, BB=BB),
        grid=(B // BB,),
        in_specs=[
            pl.BlockSpec((BB, C, G, G), lambda b: (b, 0, 0, 0)),
            pl.BlockSpec((A, 2), lambda b: (0, 0)),
        ],
        out_specs=pl.BlockSpec((BB, A, 5, G, G), lambda b: (b, 0, 0, 0, 0)),
        out_shape=jax.ShapeDtypeStruct((B, A, 5, G, G), y_pred.dtype),
    )(x_t, anchors)
    return jnp.transpose(out, (0, 1, 3, 4, 2))
